# baseline (speedup 1.0000x reference)
import functools

import jax
import jax.numpy as jnp
from jax import lax
from jax.experimental import pallas as pl
from jax.experimental.pallas import tpu as pltpu
from jax.experimental.pallas import tpu_sc as plsc

_EMB = 128
_E = 320000
_NC, _NS, _L = 2, 16, 16
_NW = _NC * _NS
_EPW = _E // _NW
_CH = 80
_KPB = 5
_BIG = _CH * _KPB
_NSLOT = _EPW // _BIG
_NGRP = _EPW // _L
_VOCAB = 64


def _build_table_body(w0_ref, w1_ref, w2_ref, t_ref):
    c = lax.broadcasted_iota(jnp.int32, (_VOCAB, 1), 0)
    oh0 = (c // 12 == lax.broadcasted_iota(jnp.int32, (_VOCAB, 5), 1)).astype(jnp.float32)
    oh1 = ((c // 2) % 6 == lax.broadcasted_iota(jnp.int32, (_VOCAB, 6), 1)).astype(jnp.float32)
    oh2 = (c % 2 == lax.broadcasted_iota(jnp.int32, (_VOCAB, 2), 1)).astype(jnp.float32)
    hi = lax.Precision.HIGHEST
    t = (jnp.dot(oh0, w0_ref[:], preferred_element_type=jnp.float32, precision=hi)
         + jnp.dot(oh1, w1_ref[:], preferred_element_type=jnp.float32, precision=hi)
         + jnp.dot(oh2, w2_ref[:], preferred_element_type=jnp.float32, precision=hi))
    for w in range(_NW):
        t_ref[pl.ds(w * _VOCAB, _VOCAB)] = t


_build_table = pl.pallas_call(
    _build_table_body,
    out_shape=jax.ShapeDtypeStruct((_NW * _VOCAB, _EMB), jnp.float32),
)


@functools.partial(
    pl.kernel,
    mesh=plsc.VectorSubcoreMesh(core_axis_name="c", subcore_axis_name="s"),
    out_type=jax.ShapeDtypeStruct((_E, _EMB), jnp.float32),
    scratch_types=[
        pltpu.VMEM((_EPW,), jnp.int32),
        pltpu.VMEM((_EPW,), jnp.int32),
        pltpu.VMEM((2, _BIG, _EMB), jnp.float32),
        pltpu.VMEM_SHARED((_NS * _VOCAB, _EMB), jnp.float32),
        pltpu.SemaphoreType.DMA,
        pltpu.SemaphoreType.DMA,
    ],
)
def _gather_kernel(a0_hbm, a1_hbm, a2_hbm, t_hbm, out_hbm,
                   col_v, idx_v, rows_v, spm, gsem, osem):
    sid = lax.axis_index("s")
    wid = sid * _NC + lax.axis_index("c")
    base = wid * _EPW
    pltpu.sync_copy(t_hbm.at[pl.ds(wid * _VOCAB, _VOCAB)],
                    spm.at[pl.ds(sid * _VOCAB, _VOCAB)])
    plsc.subcore_barrier()

    tbase = sid * _VOCAB

    def _accum(mul, add_prev):
        def grp(g, carry):
            s = pl.ds(g * _L, _L)
            v = col_v[s] * mul
            idx_v[s] = (idx_v[s] + v) if add_prev else (v + tbase)
            return carry
        lax.fori_loop(0, _NGRP, grp, 0)

    pltpu.sync_copy(a0_hbm.at[pl.ds(base, _EPW)], col_v)
    _accum(12, False)
    pltpu.sync_copy(a1_hbm.at[pl.ds(base, _EPW)], col_v)
    _accum(2, True)
    pltpu.sync_copy(a2_hbm.at[pl.ds(base, _EPW)], col_v)
    _accum(1, True)

    def fill(s, b):
        for k in range(_KPB):
            pltpu.async_copy(
                spm.at[idx_v.at[pl.ds(s * _BIG + k * _CH, _CH)]],
                rows_v.at[b].at[pl.ds(k * _CH, _CH)], gsem)

    def wait_fill(b):
        pltpu.make_async_copy(out_hbm.at[pl.ds(0, _BIG)], rows_v.at[b],
                              gsem).wait()

    def start_out(s, b):
        pltpu.async_copy(rows_v.at[b], out_hbm.at[pl.ds(base + s * _BIG, _BIG)],
                         osem)

    def wait_out(b):
        pltpu.make_async_copy(rows_v.at[b], out_hbm.at[pl.ds(0, _BIG)],
                              osem).wait()

    def step(s, b, first, last):
        wait_fill(b)
        start_out(s, b)
        if not last:
            if not first:
                wait_out(1 - b)
            fill(s + 1, 1 - b)

    fill(0, 0)
    step(0, 0, True, False)
    def body(o, carry):
        step(2 * o + 1, 1, False, False)
        step(2 * o + 2, 0, False, False)
        return carry
    lax.fori_loop(0, (_NSLOT - 3) // 2, body, 0)
    step(_NSLOT - 2, 1, False, False)
    step(_NSLOT - 1, 0, False, True)
    wait_out(1)
    wait_out(0)


def kernel(edge_attr, W0, W1, W2):
    table = _build_table(W0, W1, W2)
    a0 = edge_attr[:, 0]
    a1 = edge_attr[:, 1]
    a2 = edge_attr[:, 2]
    return _gather_kernel(a0, a1, a2, table)


# speedup vs baseline: 1.0296x; 1.0296x over previous
import functools

import jax
import jax.numpy as jnp
from jax import lax
from jax.experimental import pallas as pl
from jax.experimental.pallas import tpu as pltpu
from jax.experimental.pallas import tpu_sc as plsc

_EMB = 128
_E = 320000
_NC, _NS, _L = 2, 16, 16
_NW = _NC * _NS
_EPW = _E // _NW
_CH = 80
_NCHUNK = _EPW // _CH
_NB = 10
_G = 4
_NGRP = _EPW // _L
_VOCAB = 64


def _build_table_body(w0_ref, w1_ref, w2_ref, t_ref):
    c = lax.broadcasted_iota(jnp.int32, (_VOCAB, 1), 0)
    oh0 = (c // 12 == lax.broadcasted_iota(jnp.int32, (_VOCAB, 5), 1)).astype(jnp.float32)
    oh1 = ((c // 2) % 6 == lax.broadcasted_iota(jnp.int32, (_VOCAB, 6), 1)).astype(jnp.float32)
    oh2 = (c % 2 == lax.broadcasted_iota(jnp.int32, (_VOCAB, 2), 1)).astype(jnp.float32)
    hi = lax.Precision.HIGHEST
    t = (jnp.dot(oh0, w0_ref[:], preferred_element_type=jnp.float32, precision=hi)
         + jnp.dot(oh1, w1_ref[:], preferred_element_type=jnp.float32, precision=hi)
         + jnp.dot(oh2, w2_ref[:], preferred_element_type=jnp.float32, precision=hi))
    for w in range(_NW):
        t_ref[pl.ds(w * _VOCAB, _VOCAB)] = t


_build_table = pl.pallas_call(
    _build_table_body,
    out_shape=jax.ShapeDtypeStruct((_NW * _VOCAB, _EMB), jnp.float32),
)


@functools.partial(
    pl.kernel,
    mesh=plsc.VectorSubcoreMesh(core_axis_name="c", subcore_axis_name="s"),
    out_type=jax.ShapeDtypeStruct((_E, _EMB), jnp.float32),
    scratch_types=[
        pltpu.VMEM((_EPW,), jnp.int32),
        pltpu.VMEM((_EPW,), jnp.int32),
        pltpu.VMEM((_NB, _CH, _EMB), jnp.float32),
        pltpu.VMEM_SHARED((_NS * _VOCAB, _EMB), jnp.float32),
        pltpu.SemaphoreType.DMA,
        pltpu.SemaphoreType.DMA,
    ],
)
def _gather_kernel(a0_hbm, a1_hbm, a2_hbm, t_hbm, out_hbm,
                   col_v, idx_v, rows_v, spm, gsem, osem):
    sid = lax.axis_index("s")
    wid = sid * _NC + lax.axis_index("c")
    base = wid * _EPW
    pltpu.sync_copy(t_hbm.at[pl.ds(wid * _VOCAB, _VOCAB)],
                    spm.at[pl.ds(sid * _VOCAB, _VOCAB)])
    plsc.subcore_barrier()

    tbase = sid * _VOCAB

    def _accum(mul, add_prev):
        def grp(g, carry):
            s = pl.ds(g * _L, _L)
            v = col_v[s] * mul
            idx_v[s] = (idx_v[s] + v) if add_prev else (v + tbase)
            return carry
        lax.fori_loop(0, _NGRP, grp, 0)

    pltpu.sync_copy(a0_hbm.at[pl.ds(base, _EPW)], col_v)
    _accum(12, False)
    pltpu.sync_copy(a1_hbm.at[pl.ds(base, _EPW)], col_v)
    _accum(2, True)
    pltpu.sync_copy(a2_hbm.at[pl.ds(base, _EPW)], col_v)
    _accum(1, True)

    def start_gather(c, b):
        pltpu.async_copy(spm.at[idx_v.at[pl.ds(c * _CH, _CH)]],
                         rows_v.at[b], gsem)

    def wait_gather(b):
        pltpu.make_async_copy(out_hbm.at[pl.ds(0, _CH)], rows_v.at[b],
                              gsem).wait()

    def start_out(c, b):
        pltpu.async_copy(rows_v.at[b], out_hbm.at[pl.ds(base + c * _CH, _CH)],
                         osem)

    def wait_out(b):
        pltpu.make_async_copy(rows_v.at[b], out_hbm.at[pl.ds(0, _CH)],
                              osem).wait()

    def step(j, b):
        bn = (b + _G) % _NB
        wait_out(bn)
        start_gather(j + _G, bn)
        wait_gather(b)
        start_out(j, b)

    for b in range(_G):
        start_gather(b, b)
    for j in range(_NB):
        bn = (j + _G) % _NB
        if j >= _NB - _G:
            wait_out(bn)
        start_gather(j + _G, bn)
        wait_gather(j % _NB)
        start_out(j, j % _NB)

    def body(o, carry):
        for b in range(_NB):
            step(o * _NB + b, b)
        return carry

    lax.fori_loop(1, _NCHUNK // _NB, body, 0)

    for j in range(_NB * (_NCHUNK // _NB), _NCHUNK):
        b = j % _NB
        bn = (b + _G) % _NB
        wait_out(bn)
        if j + _G < _NCHUNK:
            start_gather(j + _G, bn)
        wait_gather(b)
        start_out(j, b)
    for b in range(_NB - _G):
        wait_out(0)


def kernel(edge_attr, W0, W1, W2):
    table = _build_table(W0, W1, W2)
    a0 = edge_attr[:, 0]
    a1 = edge_attr[:, 1]
    a2 = edge_attr[:, 2]
    return _gather_kernel(a0, a1, a2, table)


# speedup vs baseline: 1.1002x; 1.0686x over previous
import functools

import jax
import jax.numpy as jnp
from jax import lax
from jax.experimental import pallas as pl
from jax.experimental.pallas import tpu as pltpu
from jax.experimental.pallas import tpu_sc as plsc

_EMB = 128
_E = 320000
_NC, _NS, _L = 2, 16, 16
_NW = _NC * _NS
_EPW = _E // _NW
_CH = 80
_NCHUNK = _EPW // _CH
_NB = 8
_G = 3
_NGRP = _EPW // _L
_VOCAB = 64


def _build_table_body(w0_ref, w1_ref, w2_ref, t_ref):
    c = lax.broadcasted_iota(jnp.int32, (_VOCAB, 1), 0)
    oh0 = (c // 12 == lax.broadcasted_iota(jnp.int32, (_VOCAB, 5), 1)).astype(jnp.float32)
    oh1 = ((c // 2) % 6 == lax.broadcasted_iota(jnp.int32, (_VOCAB, 6), 1)).astype(jnp.float32)
    oh2 = (c % 2 == lax.broadcasted_iota(jnp.int32, (_VOCAB, 2), 1)).astype(jnp.float32)
    hi = lax.Precision.HIGHEST
    t = (jnp.dot(oh0, w0_ref[:], preferred_element_type=jnp.float32, precision=hi)
         + jnp.dot(oh1, w1_ref[:], preferred_element_type=jnp.float32, precision=hi)
         + jnp.dot(oh2, w2_ref[:], preferred_element_type=jnp.float32, precision=hi))
    for w in range(_NW):
        t_ref[pl.ds(w * _VOCAB, _VOCAB)] = t


_build_table = pl.pallas_call(
    _build_table_body,
    out_shape=jax.ShapeDtypeStruct((_NW * _VOCAB, _EMB), jnp.float32),
)


@functools.partial(
    pl.kernel,
    mesh=plsc.VectorSubcoreMesh(core_axis_name="c", subcore_axis_name="s"),
    out_type=jax.ShapeDtypeStruct((_E, _EMB), jnp.float32),
    scratch_types=[
        pltpu.VMEM((_EPW,), jnp.int32),
        pltpu.VMEM((_EPW,), jnp.int32),
        pltpu.VMEM((_EPW,), jnp.int32),
        pltpu.VMEM((_EPW,), jnp.int32),
        pltpu.VMEM((_NB, _CH, _EMB), jnp.float32),
        pltpu.VMEM_SHARED((_NS * _VOCAB, _EMB), jnp.float32),
        pltpu.SemaphoreType.DMA,
        pltpu.SemaphoreType.DMA,
    ],
)
def _gather_kernel(a0_hbm, a1_hbm, a2_hbm, t_hbm, out_hbm,
                   a0_v, a1_v, a2_v, idx_v, rows_v, spm, gsem, osem):
    sid = lax.axis_index("s")
    wid = sid * _NC + lax.axis_index("c")
    base = wid * _EPW
    pltpu.sync_copy(t_hbm.at[pl.ds(wid * _VOCAB, _VOCAB)],
                    spm.at[pl.ds(sid * _VOCAB, _VOCAB)])
    plsc.subcore_barrier()
    pltpu.sync_copy(a0_hbm.at[pl.ds(base, _EPW)], a0_v)
    pltpu.sync_copy(a1_hbm.at[pl.ds(base, _EPW)], a1_v)
    pltpu.sync_copy(a2_hbm.at[pl.ds(base, _EPW)], a2_v)

    tbase = sid * _VOCAB

    def grp(g, carry):
        s = pl.ds(g * _L, _L)
        idx_v[s] = a0_v[s] * 12 + a1_v[s] * 2 + a2_v[s] + tbase
        return carry

    lax.fori_loop(0, _NGRP, grp, 0)

    def start_gather(c, b):
        pltpu.async_copy(spm.at[idx_v.at[pl.ds(c * _CH, _CH)]],
                         rows_v.at[b], gsem)

    def wait_gather(b):
        pltpu.make_async_copy(out_hbm.at[pl.ds(0, _CH)], rows_v.at[b],
                              gsem).wait()

    def start_out(c, b):
        pltpu.async_copy(rows_v.at[b], out_hbm.at[pl.ds(base + c * _CH, _CH)],
                         osem)

    def wait_out(b):
        pltpu.make_async_copy(rows_v.at[b], out_hbm.at[pl.ds(0, _CH)],
                              osem).wait()

    def step(j, b):
        bn = (b + _G) % _NB
        wait_out(bn)
        start_gather(j + _G, bn)
        wait_gather(b)
        start_out(j, b)

    for b in range(_G):
        start_gather(b, b)
    for j in range(_NB):
        bn = (j + _G) % _NB
        if j >= _NB - _G:
            wait_out(bn)
        start_gather(j + _G, bn)
        wait_gather(j % _NB)
        start_out(j, j % _NB)

    def body(o, carry):
        for b in range(_NB):
            step(o * _NB + b, b)
        return carry

    lax.fori_loop(1, _NCHUNK // _NB, body, 0)

    for j in range(_NB * (_NCHUNK // _NB), _NCHUNK):
        b = j % _NB
        bn = (b + _G) % _NB
        wait_out(bn)
        if j + _G < _NCHUNK:
            start_gather(j + _G, bn)
        wait_gather(b)
        start_out(j, b)
    for b in range(_NB - _G):
        wait_out(0)


def kernel(edge_attr, W0, W1, W2):
    table = _build_table(W0, W1, W2)
    a0 = edge_attr[:, 0]
    a1 = edge_attr[:, 1]
    a2 = edge_attr[:, 2]
    return _gather_kernel(a0, a1, a2, table)


# speedup vs baseline: 1.1432x; 1.0391x over previous
import functools

import jax
import jax.numpy as jnp
from jax import lax
from jax.experimental import pallas as pl
from jax.experimental.pallas import tpu as pltpu
from jax.experimental.pallas import tpu_sc as plsc

_EMB = 128
_E = 320000
_NC, _NS, _L = 2, 16, 16
_NW = _NC * _NS
_EPW = _E // _NW
_CH = 80
_NCHUNK = _EPW // _CH
_NB = 8
_G = 3
_NGRP = _EPW // _L
_VOCAB = 64


def _build_table_body(w0_ref, w1_ref, w2_ref, t_ref):
    c = lax.broadcasted_iota(jnp.int32, (_VOCAB, 1), 0)
    oh0 = (c // 12 == lax.broadcasted_iota(jnp.int32, (_VOCAB, 5), 1)).astype(jnp.float32)
    oh1 = ((c // 2) % 6 == lax.broadcasted_iota(jnp.int32, (_VOCAB, 6), 1)).astype(jnp.float32)
    oh2 = (c % 2 == lax.broadcasted_iota(jnp.int32, (_VOCAB, 2), 1)).astype(jnp.float32)
    hi = lax.Precision.HIGHEST
    t = (jnp.dot(oh0, w0_ref[:], preferred_element_type=jnp.float32, precision=hi)
         + jnp.dot(oh1, w1_ref[:], preferred_element_type=jnp.float32, precision=hi)
         + jnp.dot(oh2, w2_ref[:], preferred_element_type=jnp.float32, precision=hi))
    for w in range(_NW):
        t_ref[pl.ds(w * _VOCAB, _VOCAB)] = t


_build_table = pl.pallas_call(
    _build_table_body,
    out_shape=jax.ShapeDtypeStruct((_NW * _VOCAB, _EMB), jnp.float32),
)


@functools.partial(
    pl.kernel,
    mesh=plsc.VectorSubcoreMesh(core_axis_name="c", subcore_axis_name="s"),
    out_type=jax.ShapeDtypeStruct((_E, _EMB), jnp.float32),
    scratch_types=[
        pltpu.VMEM((_EPW,), jnp.int32),
        pltpu.VMEM((_EPW,), jnp.int32),
        pltpu.VMEM((_EPW,), jnp.int32),
        pltpu.VMEM((_EPW,), jnp.int32),
        pltpu.VMEM((_NB, _CH, _EMB), jnp.float32),
        pltpu.VMEM_SHARED((_NS * _VOCAB, _EMB), jnp.float32),
        pltpu.SemaphoreType.DMA,
        pltpu.SemaphoreType.DMA,
    ],
)
def _gather_kernel(a0_hbm, a1_hbm, a2_hbm, t_hbm, out_hbm,
                   a0_v, a1_v, a2_v, idx_v, rows_v, spm, gsem, osem):
    sid = lax.axis_index("s")
    wid = sid * _NC + lax.axis_index("c")
    base = wid * _EPW
    pltpu.async_copy(a0_hbm.at[pl.ds(base, _EPW)], a0_v, gsem)
    pltpu.async_copy(a1_hbm.at[pl.ds(base, _EPW)], a1_v, gsem)
    pltpu.async_copy(a2_hbm.at[pl.ds(base, _EPW)], a2_v, gsem)
    pltpu.sync_copy(t_hbm.at[pl.ds(wid * _VOCAB, _VOCAB)],
                    spm.at[pl.ds(sid * _VOCAB, _VOCAB)])
    plsc.subcore_barrier()
    pltpu.make_async_copy(a0_hbm.at[pl.ds(base, _EPW)], a0_v, gsem).wait()
    pltpu.make_async_copy(a1_hbm.at[pl.ds(base, _EPW)], a1_v, gsem).wait()
    pltpu.make_async_copy(a2_hbm.at[pl.ds(base, _EPW)], a2_v, gsem).wait()

    tbase = sid * _VOCAB

    def compute_idx(c):
        for k in range(_CH // _L):
            s = pl.ds(c * _CH + k * _L, _L)
            idx_v[s] = a0_v[s] * 12 + a1_v[s] * 2 + a2_v[s] + tbase

    def start_gather(c, b):
        compute_idx(c)
        pltpu.async_copy(spm.at[idx_v.at[pl.ds(c * _CH, _CH)]],
                         rows_v.at[b], gsem)

    def wait_gather(b):
        pltpu.make_async_copy(out_hbm.at[pl.ds(0, _CH)], rows_v.at[b],
                              gsem).wait()

    def start_out(c, b):
        pltpu.async_copy(rows_v.at[b], out_hbm.at[pl.ds(base + c * _CH, _CH)],
                         osem)

    def wait_out(b):
        pltpu.make_async_copy(rows_v.at[b], out_hbm.at[pl.ds(0, _CH)],
                              osem).wait()

    def step(j, b):
        bn = (b + _G) % _NB
        wait_out(bn)
        start_gather(j + _G, bn)
        wait_gather(b)
        start_out(j, b)

    for b in range(_G):
        start_gather(b, b)
    for j in range(_NB):
        bn = (j + _G) % _NB
        if j >= _NB - _G:
            wait_out(bn)
        start_gather(j + _G, bn)
        wait_gather(j % _NB)
        start_out(j, j % _NB)

    def body(o, carry):
        for b in range(_NB):
            step(o * _NB + b, b)
        return carry

    lax.fori_loop(1, _NCHUNK // _NB, body, 0)

    for j in range(_NB * (_NCHUNK // _NB), _NCHUNK):
        b = j % _NB
        bn = (b + _G) % _NB
        wait_out(bn)
        if j + _G < _NCHUNK:
            start_gather(j + _G, bn)
        wait_gather(b)
        start_out(j, b)
    for b in range(_NB - _G):
        wait_out(0)


def kernel(edge_attr, W0, W1, W2):
    table = _build_table(W0, W1, W2)
    a0 = edge_attr[:, 0]
    a1 = edge_attr[:, 1]
    a2 = edge_attr[:, 2]
    return _gather_kernel(a0, a1, a2, table)


# speedup vs baseline: 1.1450x; 1.0016x over previous
import functools

import jax
import jax.numpy as jnp
from jax import lax
from jax.experimental import pallas as pl
from jax.experimental.pallas import tpu as pltpu
from jax.experimental.pallas import tpu_sc as plsc

_EMB = 128
_E = 320000
_NC, _NS, _L = 2, 16, 16
_NW = _NC * _NS
_EPW = _E // _NW
_CH = 80
_NCHUNK = _EPW // _CH
_NB = 8
_G = 4
_NGRP = _EPW // _L
_VOCAB = 64


def _build_table_body(w0_ref, w1_ref, w2_ref, t_ref):
    c = lax.broadcasted_iota(jnp.int32, (_VOCAB, 1), 0)
    oh0 = (c // 12 == lax.broadcasted_iota(jnp.int32, (_VOCAB, 5), 1)).astype(jnp.float32)
    oh1 = ((c // 2) % 6 == lax.broadcasted_iota(jnp.int32, (_VOCAB, 6), 1)).astype(jnp.float32)
    oh2 = (c % 2 == lax.broadcasted_iota(jnp.int32, (_VOCAB, 2), 1)).astype(jnp.float32)
    hi = lax.Precision.HIGHEST
    t = (jnp.dot(oh0, w0_ref[:], preferred_element_type=jnp.float32, precision=hi)
         + jnp.dot(oh1, w1_ref[:], preferred_element_type=jnp.float32, precision=hi)
         + jnp.dot(oh2, w2_ref[:], preferred_element_type=jnp.float32, precision=hi))
    for w in range(_NW):
        t_ref[pl.ds(w * _VOCAB, _VOCAB)] = t


_build_table = pl.pallas_call(
    _build_table_body,
    out_shape=jax.ShapeDtypeStruct((_NW * _VOCAB, _EMB), jnp.float32),
)


@functools.partial(
    pl.kernel,
    mesh=plsc.VectorSubcoreMesh(core_axis_name="c", subcore_axis_name="s"),
    out_type=jax.ShapeDtypeStruct((_E, _EMB), jnp.float32),
    scratch_types=[
        pltpu.VMEM((_EPW,), jnp.int32),
        pltpu.VMEM((_EPW,), jnp.int32),
        pltpu.VMEM((_EPW,), jnp.int32),
        pltpu.VMEM((_EPW,), jnp.int32),
        pltpu.VMEM((_NB, _CH, _EMB), jnp.float32),
        pltpu.VMEM_SHARED((_NS * _VOCAB, _EMB), jnp.float32),
        pltpu.SemaphoreType.DMA,
        pltpu.SemaphoreType.DMA,
    ],
)
def _gather_kernel(a0_hbm, a1_hbm, a2_hbm, t_hbm, out_hbm,
                   a0_v, a1_v, a2_v, idx_v, rows_v, spm, gsem, osem):
    sid = lax.axis_index("s")
    wid = sid * _NC + lax.axis_index("c")
    base = wid * _EPW
    pltpu.async_copy(a0_hbm.at[pl.ds(base, _EPW)], a0_v, gsem)
    pltpu.async_copy(a1_hbm.at[pl.ds(base, _EPW)], a1_v, gsem)
    pltpu.async_copy(a2_hbm.at[pl.ds(base, _EPW)], a2_v, gsem)
    pltpu.sync_copy(t_hbm.at[pl.ds(wid * _VOCAB, _VOCAB)],
                    spm.at[pl.ds(sid * _VOCAB, _VOCAB)])
    plsc.subcore_barrier()
    pltpu.make_async_copy(a0_hbm.at[pl.ds(base, _EPW)], a0_v, gsem).wait()
    pltpu.make_async_copy(a1_hbm.at[pl.ds(base, _EPW)], a1_v, gsem).wait()
    pltpu.make_async_copy(a2_hbm.at[pl.ds(base, _EPW)], a2_v, gsem).wait()

    tbase = sid * _VOCAB

    def compute_idx(c):
        for k in range(_CH // _L):
            s = pl.ds(c * _CH + k * _L, _L)
            idx_v[s] = a0_v[s] * 12 + a1_v[s] * 2 + a2_v[s] + tbase

    def start_gather(c, b):
        compute_idx(c)
        pltpu.async_copy(spm.at[idx_v.at[pl.ds(c * _CH, _CH)]],
                         rows_v.at[b], gsem)

    def wait_gather(b):
        pltpu.make_async_copy(out_hbm.at[pl.ds(0, _CH)], rows_v.at[b],
                              gsem).wait()

    def start_out(c, b):
        pltpu.async_copy(rows_v.at[b], out_hbm.at[pl.ds(base + c * _CH, _CH)],
                         osem)

    def wait_out(b):
        pltpu.make_async_copy(rows_v.at[b], out_hbm.at[pl.ds(0, _CH)],
                              osem).wait()

    def step(j, b):
        bn = (b + _G) % _NB
        wait_out(bn)
        start_gather(j + _G, bn)
        wait_gather(b)
        start_out(j, b)

    for b in range(_G):
        start_gather(b, b)
    for j in range(_NB):
        bn = (j + _G) % _NB
        if j >= _NB - _G:
            wait_out(bn)
        start_gather(j + _G, bn)
        wait_gather(j % _NB)
        start_out(j, j % _NB)

    def body(o, carry):
        for b in range(_NB):
            step(o * _NB + b, b)
        return carry

    lax.fori_loop(1, _NCHUNK // _NB, body, 0)

    for j in range(_NB * (_NCHUNK // _NB), _NCHUNK):
        b = j % _NB
        bn = (b + _G) % _NB
        wait_out(bn)
        if j + _G < _NCHUNK:
            start_gather(j + _G, bn)
        wait_gather(b)
        start_out(j, b)
    for b in range(_NB - _G):
        wait_out(0)


def kernel(edge_attr, W0, W1, W2):
    table = _build_table(W0, W1, W2)
    a0 = edge_attr[:, 0]
    a1 = edge_attr[:, 1]
    a2 = edge_attr[:, 2]
    return _gather_kernel(a0, a1, a2, table)
